# initial kernel scaffold (unmeasured)
import jax
import jax.numpy as jnp
from jax import lax
from jax.experimental import pallas as pl
from jax.experimental.pallas import tpu as pltpu

N_DEV = 8


def kernel(x, w_mat, scale_x, scale_w):
    m_per, k = x.shape
    k2, n_per = w_mat.shape
    assert k == k2

    def body(x_ref, w_ref, sx_ref, sw_ref, out_ref,
             x8_ref, w_bf_ref, comm_ref, send_sems, recv_sems):
        my = lax.axis_index("i")
        left = lax.rem(my + N_DEV - 1, N_DEV)
        right = lax.rem(my + 1, N_DEV)

        x8_ref[...] = x_ref[...].astype(jnp.float8_e5m2)
        w_bf_ref[...] = w_ref[...].astype(jnp.bfloat16)

        barrier_sem = pltpu.get_barrier_semaphore()
        for nbr in (left, right):
            pl.semaphore_signal(
                barrier_sem, inc=1,
                device_id=(nbr,), device_id_type=pl.DeviceIdType.MESH,
            )
        pl.semaphore_wait(barrier_sem, 2)

        s = sx_ref[0] * sw_ref[0]

        def gemm_store(chunk_f8, origin):
            acc = jnp.dot(chunk_f8.astype(jnp.bfloat16), w_bf_ref[...],
                          preferred_element_type=jnp.float32)
            y = acc * s
            z = jnp.clip(y, -60.0, 60.0)
            out_ref[pl.ds(origin * m_per, m_per), :] = y / (1.0 + jnp.exp(-z))

        prev_origin = my
        for h in range(N_DEV - 1):
            src = x8_ref if h == 0 else comm_ref.at[h - 1]
            rdma = pltpu.make_async_remote_copy(
                src_ref=src,
                dst_ref=comm_ref.at[h],
                send_sem=send_sems.at[h],
                recv_sem=recv_sems.at[h],
                device_id=(right,),
                device_id_type=pl.DeviceIdType.MESH,
            )
            rdma.start()
            chunk = x8_ref[...] if h == 0 else comm_ref[h - 1]
            gemm_store(chunk, prev_origin)
            rdma.wait()
            prev_origin = lax.rem(my - (h + 1) + N_DEV, N_DEV)
        gemm_store(comm_ref[N_DEV - 2], prev_origin)

    return pl.pallas_call(
        body,
        out_shape=jax.ShapeDtypeStruct((N_DEV * m_per, n_per), jnp.float32),
        in_specs=[
            pl.BlockSpec(memory_space=pltpu.VMEM),
            pl.BlockSpec(memory_space=pltpu.VMEM),
            pl.BlockSpec(memory_space=pltpu.SMEM),
            pl.BlockSpec(memory_space=pltpu.SMEM),
        ],
        out_specs=pl.BlockSpec(memory_space=pltpu.VMEM),
        scratch_shapes=[
            pltpu.VMEM((m_per, k), jnp.float8_e5m2),
            pltpu.VMEM((k, n_per), jnp.bfloat16),
            pltpu.VMEM((N_DEV - 1, m_per, k), jnp.float8_e5m2),
            pltpu.SemaphoreType.DMA((N_DEV - 1,)),
            pltpu.SemaphoreType.DMA((N_DEV - 1,)),
        ],
        compiler_params=pltpu.CompilerParams(collective_id=0),
    )(x, w_mat, scale_x, scale_w)


# baseline (device time: 203988 ns/iter reference)
import jax
import jax.numpy as jnp
from jax import lax
from jax.experimental import pallas as pl
from jax.experimental.pallas import tpu as pltpu

N_DEV = 8


def kernel(x, w_mat, scale_x, scale_w):
    m_per, k = x.shape
    k2, n_per = w_mat.shape
    assert k == k2

    x8 = x.astype(jnp.float8_e5m2)
    w_bf = w_mat.astype(jnp.bfloat16)

    def body(x_ref, w_ref, sx_ref, sw_ref, out_ref,
             comm_ref, stage_ref, send_sems, recv_sems, out_sems):
        my = lax.axis_index("i")
        left = lax.rem(my + N_DEV - 1, N_DEV)
        right = lax.rem(my + 1, N_DEV)

        barrier_sem = pltpu.get_barrier_semaphore()
        for nbr in (left, right):
            pl.semaphore_signal(
                barrier_sem, inc=1,
                device_id=(nbr,), device_id_type=pl.DeviceIdType.MESH,
            )
        pl.semaphore_wait(barrier_sem, 2)

        s = sx_ref[0] * sw_ref[0]
        out_copies = [None, None]

        def gemm_store(chunk_f8, origin, slot):
            acc = jnp.dot(chunk_f8.astype(jnp.bfloat16), w_ref[...],
                          preferred_element_type=jnp.float32)
            y = acc * s
            z = jnp.clip(y, -60.0, 60.0)
            if out_copies[slot] is not None:
                out_copies[slot].wait()
            stage_ref[slot] = y / (1.0 + jnp.exp(-z))
            cp = pltpu.make_async_copy(
                stage_ref.at[slot],
                out_ref.at[pl.ds(origin * m_per, m_per), :],
                out_sems.at[slot],
            )
            cp.start()
            out_copies[slot] = cp

        prev_origin = my
        for h in range(N_DEV - 1):
            src = x_ref if h == 0 else comm_ref.at[h - 1]
            rdma = pltpu.make_async_remote_copy(
                src_ref=src,
                dst_ref=comm_ref.at[h],
                send_sem=send_sems.at[h],
                recv_sem=recv_sems.at[h],
                device_id=(right,),
                device_id_type=pl.DeviceIdType.MESH,
            )
            rdma.start()
            chunk = x_ref[...] if h == 0 else comm_ref[h - 1]
            gemm_store(chunk, prev_origin, h % 2)
            rdma.wait()
            prev_origin = lax.rem(my - (h + 1) + N_DEV, N_DEV)
        gemm_store(comm_ref[N_DEV - 2], prev_origin, (N_DEV - 1) % 2)
        for cp in out_copies:
            cp.wait()

    return pl.pallas_call(
        body,
        out_shape=jax.ShapeDtypeStruct((N_DEV * m_per, n_per), jnp.float32),
        in_specs=[
            pl.BlockSpec(memory_space=pltpu.VMEM),
            pl.BlockSpec(memory_space=pltpu.VMEM),
            pl.BlockSpec(memory_space=pltpu.SMEM),
            pl.BlockSpec(memory_space=pltpu.SMEM),
        ],
        out_specs=pl.BlockSpec(memory_space=pl.ANY),
        scratch_shapes=[
            pltpu.VMEM((N_DEV - 1, m_per, k), jnp.float8_e5m2),
            pltpu.VMEM((2, m_per, n_per), jnp.float32),
            pltpu.SemaphoreType.DMA((N_DEV - 1,)),
            pltpu.SemaphoreType.DMA((N_DEV - 1,)),
            pltpu.SemaphoreType.DMA((2,)),
        ],
        compiler_params=pltpu.CompilerParams(collective_id=0),
    )(x8, w_bf, scale_x, scale_w)


# device time: 125902 ns/iter; 1.6202x vs baseline; 1.6202x over previous
import jax
import jax.numpy as jnp
from jax import lax
from jax.experimental import pallas as pl
from jax.experimental.pallas import tpu as pltpu

N_DEV = 8
N_HOP = N_DEV - 1


def kernel(x, w_mat, scale_x, scale_w):
    m_per, k = x.shape
    k2, n_per = w_mat.shape
    assert k == k2
    half = m_per // 2

    x8 = x.astype(jnp.float8_e5m2)
    w_bf = w_mat.astype(jnp.bfloat16)

    def body(x_ref, w_ref, sx_ref, sw_ref, out_ref,
             comm_r, comm_l, stage_ref,
             send_r, recv_r, send_l, recv_l, out_sems):
        my = lax.axis_index("i")
        left = lax.rem(my + N_DEV - 1, N_DEV)
        right = lax.rem(my + 1, N_DEV)

        barrier_sem = pltpu.get_barrier_semaphore()
        for nbr in (left, right):
            pl.semaphore_signal(
                barrier_sem, inc=1,
                device_id=(nbr,), device_id_type=pl.DeviceIdType.MESH,
            )
        pl.semaphore_wait(barrier_sem, 2)

        s = sx_ref[0] * sw_ref[0]
        out_copies = [None] * 4
        slot_ctr = [0]

        def gemm_store(chunk_f8, row_start):
            slot = slot_ctr[0] % 4
            slot_ctr[0] += 1
            acc = jnp.dot(chunk_f8.astype(jnp.bfloat16), w_ref[...],
                          preferred_element_type=jnp.float32)
            y = acc * s
            z = jnp.clip(y, -60.0, 60.0)
            if out_copies[slot] is not None:
                out_copies[slot].wait()
            stage_ref[slot] = y / (1.0 + jnp.exp(-z))
            cp = pltpu.make_async_copy(
                stage_ref.at[slot],
                out_ref.at[pl.ds(row_start, half), :],
                out_sems.at[slot],
            )
            cp.start()
            out_copies[slot] = cp

        def hop(h, direction):
            comm = comm_r if direction == 0 else comm_l
            src = (x_ref.at[pl.ds(direction * half, half), :] if h == 0
                   else comm.at[h - 1])
            return pltpu.make_async_remote_copy(
                src_ref=src,
                dst_ref=comm.at[h],
                send_sem=(send_r if direction == 0 else send_l).at[h],
                recv_sem=(recv_r if direction == 0 else recv_l).at[h],
                device_id=(right if direction == 0 else left,),
                device_id_type=pl.DeviceIdType.MESH,
            )

        rd = hop(0, 0)
        ld = hop(0, 1)
        rd.start()
        ld.start()
        gemm_store(x_ref[pl.ds(0, half), :], my * m_per)
        gemm_store(x_ref[pl.ds(half, half), :], my * m_per + half)

        for h in range(N_HOP):
            rd.wait()
            if h + 1 < N_HOP:
                rd = hop(h + 1, 0)
                rd.start()
            origin_r = lax.rem(my - (h + 1) + N_DEV, N_DEV)
            gemm_store(comm_r[h], origin_r * m_per)

            ld.wait()
            if h + 1 < N_HOP:
                ld = hop(h + 1, 1)
                ld.start()
            origin_l = lax.rem(my + (h + 1), N_DEV)
            gemm_store(comm_l[h], origin_l * m_per + half)

        for cp in out_copies:
            cp.wait()

    return pl.pallas_call(
        body,
        out_shape=jax.ShapeDtypeStruct((N_DEV * m_per, n_per), jnp.float32),
        in_specs=[
            pl.BlockSpec(memory_space=pltpu.VMEM),
            pl.BlockSpec(memory_space=pltpu.VMEM),
            pl.BlockSpec(memory_space=pltpu.SMEM),
            pl.BlockSpec(memory_space=pltpu.SMEM),
        ],
        out_specs=pl.BlockSpec(memory_space=pl.ANY),
        scratch_shapes=[
            pltpu.VMEM((N_HOP, half, k), jnp.float8_e5m2),
            pltpu.VMEM((N_HOP, half, k), jnp.float8_e5m2),
            pltpu.VMEM((4, half, n_per), jnp.float32),
            pltpu.SemaphoreType.DMA((N_HOP,)),
            pltpu.SemaphoreType.DMA((N_HOP,)),
            pltpu.SemaphoreType.DMA((N_HOP,)),
            pltpu.SemaphoreType.DMA((N_HOP,)),
            pltpu.SemaphoreType.DMA((4,)),
        ],
        compiler_params=pltpu.CompilerParams(collective_id=0),
    )(x8, w_bf, scale_x, scale_w)


# device time: 103188 ns/iter; 1.9769x vs baseline; 1.2201x over previous
import jax
import jax.numpy as jnp
from jax import lax
from jax.experimental import pallas as pl
from jax.experimental.pallas import tpu as pltpu

N_DEV = 8
N_HOP = N_DEV - 1
N_SEG = 2
W_SLICES = 8


def kernel(x, w_mat, scale_x, scale_w):
    m_per, k = x.shape
    k2, n_per = w_mat.shape
    assert k == k2
    half = m_per // 2
    seg = half // N_SEG
    ws = k // W_SLICES

    x8 = x.astype(jnp.float8_e5m2)

    def body(x_ref, w_hbm, sx_ref, sw_ref, out_ref,
             w_bf, wbuf, comm_r, comm_l, stage_ref,
             send_r, recv_r, send_l, recv_l, wsems, out_sems):
        my = lax.axis_index("i")
        left = lax.rem(my + N_DEV - 1, N_DEV)
        right = lax.rem(my + 1, N_DEV)

        barrier_sem = pltpu.get_barrier_semaphore()
        for nbr in (left, right):
            pl.semaphore_signal(
                barrier_sem, inc=1,
                device_id=(nbr,), device_id_type=pl.DeviceIdType.MESH,
            )
        pl.semaphore_wait(barrier_sem, 2)

        def mk(d, h, s):
            comm = comm_r if d == 0 else comm_l
            if h == 0:
                src = x_ref.at[pl.ds(d * half + s * seg, seg), :]
            else:
                src = comm.at[h - 1, pl.ds(s * seg, seg), :]
            return pltpu.make_async_remote_copy(
                src_ref=src,
                dst_ref=comm.at[h, pl.ds(s * seg, seg), :],
                send_sem=(send_r if d == 0 else send_l).at[h, s],
                recv_sem=(recv_r if d == 0 else recv_l).at[h, s],
                device_id=(right if d == 0 else left,),
                device_id_type=pl.DeviceIdType.MESH,
            )

        desc = [[[mk(d, h, s) for s in range(N_SEG)] for h in range(N_HOP)]
                for d in range(2)]
        for d in range(2):
            for s in range(N_SEG):
                desc[d][0][s].start()

        wd = [pltpu.make_async_copy(
                  w_hbm.at[pl.ds(i * ws, ws), :], wbuf.at[i % 2],
                  wsems.at[i % 2])
              for i in range(W_SLICES)]
        wd[0].start()
        wd[1].start()
        for i in range(W_SLICES):
            wd[i].wait()
            w_bf[pl.ds(i * ws, ws), :] = wbuf[i % 2].astype(jnp.bfloat16)
            if i + 2 < W_SLICES:
                wd[i + 2].start()

        s_val = sx_ref[0] * sw_ref[0]
        out_copies = [None, None]
        slot_ctr = [0]

        def gemm_store(chunk_f8, row_start):
            slot = slot_ctr[0] % 2
            slot_ctr[0] += 1
            acc = jnp.dot(chunk_f8.astype(jnp.bfloat16), w_bf[...],
                          preferred_element_type=jnp.float32)
            y = acc * s_val
            z = jnp.clip(y, -60.0, 60.0)
            if out_copies[slot] is not None:
                out_copies[slot].wait()
            stage_ref[slot] = y / (1.0 + jnp.exp(-z))
            cp = pltpu.make_async_copy(
                stage_ref.at[slot],
                out_ref.at[pl.ds(row_start, half), :],
                out_sems.at[slot],
            )
            cp.start()
            out_copies[slot] = cp

        gemm_store(x_ref[pl.ds(0, half), :], my * m_per)
        gemm_store(x_ref[pl.ds(half, half), :], my * m_per + half)

        for h in range(N_HOP):
            for s in range(N_SEG):
                for d in range(2):
                    desc[d][h][s].wait_recv()
                    if h + 1 < N_HOP:
                        desc[d][h + 1][s].start()
            origin_r = lax.rem(my - (h + 1) + N_DEV, N_DEV)
            gemm_store(comm_r[h], origin_r * m_per)
            origin_l = lax.rem(my + (h + 1), N_DEV)
            gemm_store(comm_l[h], origin_l * m_per + half)

        for d in range(2):
            for h in range(N_HOP):
                for s in range(N_SEG):
                    desc[d][h][s].wait_send()
        for cp in out_copies:
            cp.wait()

    return pl.pallas_call(
        body,
        out_shape=jax.ShapeDtypeStruct((N_DEV * m_per, n_per), jnp.float32),
        in_specs=[
            pl.BlockSpec(memory_space=pltpu.VMEM),
            pl.BlockSpec(memory_space=pl.ANY),
            pl.BlockSpec(memory_space=pltpu.SMEM),
            pl.BlockSpec(memory_space=pltpu.SMEM),
        ],
        out_specs=pl.BlockSpec(memory_space=pl.ANY),
        scratch_shapes=[
            pltpu.VMEM((k, n_per), jnp.bfloat16),
            pltpu.VMEM((2, ws, n_per), jnp.float32),
            pltpu.VMEM((N_HOP, half, k), jnp.float8_e5m2),
            pltpu.VMEM((N_HOP, half, k), jnp.float8_e5m2),
            pltpu.VMEM((2, half, n_per), jnp.float32),
            pltpu.SemaphoreType.DMA((N_HOP, N_SEG)),
            pltpu.SemaphoreType.DMA((N_HOP, N_SEG)),
            pltpu.SemaphoreType.DMA((N_HOP, N_SEG)),
            pltpu.SemaphoreType.DMA((N_HOP, N_SEG)),
            pltpu.SemaphoreType.DMA((2,)),
            pltpu.SemaphoreType.DMA((2,)),
        ],
        compiler_params=pltpu.CompilerParams(collective_id=0),
    )(x8, w_mat, scale_x, scale_w)
